# baseline (speedup 1.0000x reference)
import jax
import jax.numpy as jnp
from jax import lax
from jax.experimental import pallas as pl
from jax.experimental.pallas import tpu as pltpu
from jax.experimental.pallas import tpu_sc as plsc

_VOCAB = 1000000
_EMB = 32
_BATCH = 4096
_MAXLEN = 50

_NC = 2
_NS = 16
_NW = _NC * _NS

_LANES = 16
_FULL_TILES = _VOCAB // 128
_TAIL_V0 = _FULL_TILES * 128
_COLS_PER_W = 245
_NBA = 5
_NGA = _COLS_PER_W // _NBA

_CHUNK = 128
_CHUNKS = 2 * _MAXLEN * (_BATCH // _CHUNK)
_CPW = _CHUNKS // _NW
_NBB = 10
_NGB = _CPW // _NBB
_HALF = _MAXLEN * (_BATCH // _CHUNK)


def _worker_id():
    return lax.axis_index("s") * _NC + lax.axis_index("c")


def _iota16():
    return jnp.arange(16, dtype=jnp.int32)


def _relayout_body(wt, wtail, wlin, tailv, ibuf, obuf, isem, osem):
    wid = _worker_id()

    @pl.when(wid == 0)
    def _():
        pltpu.sync_copy(wtail, tailv)
        pltpu.sync_copy(tailv, wlin.at[pl.ds(_TAIL_V0 // 4, 16)])

    base = wid * _COLS_PER_W
    iot = _iota16()
    row_even = iot
    row_odd = iot + 16

    def _transpose_tile(b):
        def srow(s, carry):
            for k in range(8):
                col = jnp.full((16,), 4 * s + k // 2, jnp.int32)
                rows = row_even if k % 2 == 0 else row_odd
                bvec = jnp.full((16,), b, jnp.int32)
                v = plsc.load_gather(ibuf, [bvec, rows, col])
                obuf[b, s, pl.ds(16 * k, 16)] = v
            return carry

        lax.fori_loop(0, 32, srow, 0)

    def group(g, carry):
        for b in range(_NBA):
            c_prev = base + (g - 1) * _NBA + b

            @pl.when((g > 0) & (c_prev < _FULL_TILES))
            def _(b=b):
                pltpu.make_async_copy(
                    obuf.at[b], wlin.at[pl.ds(0, 32)], osem
                ).wait()

        for b in range(_NBA):
            c = base + g * _NBA + b

            @pl.when(c < _FULL_TILES)
            def _(c=c, b=b):
                pltpu.async_copy(
                    wt.at[:, pl.ds(c * 128, 128)], ibuf.at[b], isem
                )
        for b in range(_NBA):
            c = base + g * _NBA + b

            @pl.when(c < _FULL_TILES)
            def _(c=c, b=b):
                pltpu.make_async_copy(
                    wt.at[:, pl.ds(c * 128, 128)], ibuf.at[b], isem
                ).wait()
        for b in range(_NBA):
            c = base + g * _NBA + b

            @pl.when(c < _FULL_TILES)
            def _(c=c, b=b):
                _transpose_tile(b)
                pltpu.async_copy(
                    obuf.at[b], wlin.at[pl.ds(c * 32, 32)], osem
                )
        return carry

    lax.fori_loop(0, _NGA, group, 0)
    for b in range(_NBA):
        c = base + (_NGA - 1) * _NBA + b

        @pl.when(c < _FULL_TILES)
        def _(c=c, b=b):
            pltpu.make_async_copy(
                obuf.at[b], wlin.at[pl.ds(0, 32)], osem
            ).wait()


_relayout = pl.kernel(
    _relayout_body,
    out_type=jax.ShapeDtypeStruct((_VOCAB // 4, 128), jnp.float32),
    mesh=plsc.VectorSubcoreMesh(core_axis_name="c", subcore_axis_name="s"),
    compiler_params=pltpu.CompilerParams(
        use_tc_tiling_on_sc=True, needs_layout_passes=False
    ),
    scratch_types=[
        pltpu.VMEM((16, 128), jnp.float32),
        pltpu.VMEM((_NBA, 32, 128), jnp.float32),
        pltpu.VMEM((_NBA, 32, 128), jnp.float32),
        pltpu.SemaphoreType.DMA,
        pltpu.SemaphoreType.DMA,
    ],
)


def _gather_body(table, idx, out0, out1, idx_v, gbuf, tbuf, gsem, wsem):
    wid = _worker_id()
    pltpu.sync_copy(idx.at[pl.ds(wid * _CPW, _CPW)], idx_v)
    iot = _iota16()
    rowv = [iot + 16 * k for k in range(8)]

    def _transpose_chunk(b):
        def erow(ee, carry):
            col = jnp.full((16,), ee, jnp.int32)
            bvec = jnp.full((16,), b, jnp.int32)
            for k in range(8):
                v = plsc.load_gather(gbuf, [bvec, rowv[k], col])
                tbuf[b, ee // 8, ee % 8, pl.ds(16 * k, 16)] = v
            return carry

        lax.fori_loop(0, 32, erow, 0)

    def _run(out, hbase):
        def group(g, carry):
            @pl.when(g > 0)
            def _():
                for b in range(_NBB):
                    pltpu.make_async_copy(
                        tbuf.at[b], out.at[0, :, 0], wsem
                    ).wait()

            for b in range(_NBB):
                jj = g * _NBB + b
                pltpu.async_copy(
                    table.at[idx_v.at[jj]], gbuf.at[b], gsem
                )
            for b in range(_NBB):
                jj = g * _NBB + b
                pltpu.make_async_copy(
                    table.at[idx_v.at[jj]], gbuf.at[b], gsem
                ).wait()
            for b in range(_NBB):
                jj = g * _NBB + b
                _transpose_chunk(b)
                h = hbase + jj
                pltpu.async_copy(
                    tbuf.at[b], out.at[h // 32, :, h % 32], wsem
                )
            return carry

        lax.fori_loop(0, _NGB, group, 0)
        for b in range(_NBB):
            pltpu.make_async_copy(tbuf.at[b], out.at[0, :, 0], wsem).wait()

    @pl.when(wid < _NS)
    def _():
        _run(out0, wid * _CPW)

    @pl.when(wid >= _NS)
    def _():
        _run(out1, wid * _CPW - _HALF)


_gather = pl.kernel(
    _gather_body,
    out_type=(
        jax.ShapeDtypeStruct((_MAXLEN, 4, 32, 8, 128), jnp.float32),
        jax.ShapeDtypeStruct((_MAXLEN, 4, 32, 8, 128), jnp.float32),
    ),
    mesh=plsc.VectorSubcoreMesh(core_axis_name="c", subcore_axis_name="s"),
    compiler_params=pltpu.CompilerParams(
        use_tc_tiling_on_sc=False, needs_layout_passes=False
    ),
    scratch_types=[
        pltpu.VMEM((_CPW, _CHUNK), jnp.int32),
        pltpu.VMEM((_NBB, _CHUNK, _EMB), jnp.float32),
        pltpu.VMEM((_NBB, 4, 8, 128), jnp.float32),
        pltpu.SemaphoreType.DMA,
        pltpu.SemaphoreType.DMA,
    ],
)


def _seqlen_body(x_ref, o_ref):
    o_ref[...] = jnp.sum(jnp.sign(x_ref[...]), axis=1, keepdims=True)


_seqlen = pl.pallas_call(
    _seqlen_body,
    out_shape=jax.ShapeDtypeStruct((_BATCH * 2, 1), jnp.int32),
)


def kernel(x, embedding_w):
    wt = embedding_w.T
    wtail = embedding_w[_TAIL_V0:].reshape(16, 128)
    wlin = _relayout(wt, wtail)
    wrm = wlin.reshape(_VOCAB, _EMB)
    xt = jnp.transpose(x, (1, 2, 0)).reshape(_CHUNKS, _CHUNK)
    o0, o1 = _gather(wrm, xt)
    e_hypo = o0.transpose(2, 4, 0, 1, 3).reshape(_BATCH, _MAXLEN, _EMB)
    e_prem = o1.transpose(2, 4, 0, 1, 3).reshape(_BATCH, _MAXLEN, _EMB)
    seq = _seqlen(x.reshape(_BATCH * 2, _MAXLEN)).reshape(_BATCH, 2)
    return (e_hypo, e_prem, seq[:, 0], seq[:, 1])


# speedup vs baseline: 1.4262x; 1.4262x over previous
import jax
import jax.numpy as jnp
from jax import lax
from jax.experimental import pallas as pl
from jax.experimental.pallas import tpu as pltpu
from jax.experimental.pallas import tpu_sc as plsc

_VOCAB = 1000000
_EMB = 32
_BATCH = 4096
_MAXLEN = 50

_NC = 2
_NS = 16
_NW = _NC * _NS

_CHUNK = 128
_CHUNKS = 2 * _MAXLEN * (_BATCH // _CHUNK)
_CPW = _CHUNKS // _NW
_NBB = 10
_NGB = _CPW // _NBB
_HALF = _MAXLEN * (_BATCH // _CHUNK)


def _worker_id():
    return lax.axis_index("s") * _NC + lax.axis_index("c")


def _gather_body(table, idx, out0, out1, idx_v, gbuf, tbuf, gsem, wsem):
    wid = _worker_id()
    pltpu.sync_copy(idx.at[pl.ds(wid * _CPW, _CPW)], idx_v)
    iot = jnp.arange(16, dtype=jnp.int32)

    def _transpose_chunk(b):
        rowv = [iot + (b * _CHUNK + 16 * k) for k in range(8)]

        def erow(ee, carry):
            col = jnp.full((16,), ee, jnp.int32)
            for k in range(8):
                v = plsc.load_gather(gbuf, [rowv[k], col])
                tbuf[b, ee // 8, ee % 8, pl.ds(16 * k, 16)] = v
            return carry

        lax.fori_loop(0, 32, erow, 0)

    def _run(out, hbase):
        def group(g, carry):
            @pl.when(g > 0)
            def _():
                for b in range(_NBB):
                    pltpu.make_async_copy(
                        tbuf.at[b], out.at[0, :, 0], wsem
                    ).wait()

            for b in range(_NBB):
                jj = g * _NBB + b
                pltpu.async_copy(
                    table.at[idx_v.at[jj]],
                    gbuf.at[pl.ds(b * _CHUNK, _CHUNK)],
                    gsem,
                )
            for b in range(_NBB):
                jj = g * _NBB + b
                pltpu.make_async_copy(
                    table.at[idx_v.at[jj]],
                    gbuf.at[pl.ds(b * _CHUNK, _CHUNK)],
                    gsem,
                ).wait()
            for b in range(_NBB):
                jj = g * _NBB + b
                _transpose_chunk(b)
                h = hbase + jj
                pltpu.async_copy(
                    tbuf.at[b], out.at[h // 32, :, h % 32], wsem
                )
            return carry

        lax.fori_loop(0, _NGB, group, 0)
        for b in range(_NBB):
            pltpu.make_async_copy(tbuf.at[b], out.at[0, :, 0], wsem).wait()

    @pl.when(wid < _NS)
    def _():
        _run(out0, wid * _CPW)

    @pl.when(wid >= _NS)
    def _():
        _run(out1, wid * _CPW - _HALF)


_gather = pl.kernel(
    _gather_body,
    out_type=(
        jax.ShapeDtypeStruct((_MAXLEN, 4, 32, 8, 128), jnp.float32),
        jax.ShapeDtypeStruct((_MAXLEN, 4, 32, 8, 128), jnp.float32),
    ),
    mesh=plsc.VectorSubcoreMesh(core_axis_name="c", subcore_axis_name="s"),
    compiler_params=pltpu.CompilerParams(
        use_tc_tiling_on_sc=False, needs_layout_passes=False
    ),
    scratch_types=[
        pltpu.VMEM((_CPW, _CHUNK), jnp.int32),
        pltpu.VMEM((_NBB * _CHUNK, _EMB), jnp.float32),
        pltpu.VMEM((_NBB, 4, 8, 128), jnp.float32),
        pltpu.SemaphoreType.DMA,
        pltpu.SemaphoreType.DMA,
    ],
)


def _seqlen_body(x_ref, o_ref):
    o_ref[...] = jnp.sum(jnp.sign(x_ref[...]), axis=1, keepdims=True)


_seqlen = pl.pallas_call(
    _seqlen_body,
    out_shape=jax.ShapeDtypeStruct((_BATCH * 2, 1), jnp.int32),
)


def kernel(x, embedding_w):
    wpack = lax.optimization_barrier(embedding_w.reshape(_VOCAB // 4, 128))
    wrm = wpack.reshape(_VOCAB, _EMB)
    xt = jnp.transpose(x, (1, 2, 0)).reshape(_CHUNKS, _CHUNK)
    o0, o1 = _gather(wrm, xt)
    e_hypo = o0.transpose(2, 4, 0, 1, 3).reshape(_BATCH, _MAXLEN, _EMB)
    e_prem = o1.transpose(2, 4, 0, 1, 3).reshape(_BATCH, _MAXLEN, _EMB)
    seq = _seqlen(x.reshape(_BATCH * 2, _MAXLEN)).reshape(_BATCH, 2)
    return (e_hypo, e_prem, seq[:, 0], seq[:, 1])


# speedup vs baseline: 1.4275x; 1.0009x over previous
import jax
import jax.numpy as jnp
from jax import lax
from jax.experimental import pallas as pl
from jax.experimental.pallas import tpu as pltpu
from jax.experimental.pallas import tpu_sc as plsc

_VOCAB = 1000000
_EMB = 32
_BATCH = 4096
_MAXLEN = 50

_NC = 2
_NS = 16
_NW = _NC * _NS

_CHUNK = 128
_CHUNKS = 2 * _MAXLEN * (_BATCH // _CHUNK)
_CPW = _CHUNKS // _NW
_NBB = 10
_NGB = _CPW // _NBB
_HALF = _MAXLEN * (_BATCH // _CHUNK)


def _worker_id():
    return lax.axis_index("s") * _NC + lax.axis_index("c")


def _gather_body(table, idx, out0, out1, idx_v, gbuf, tbuf, gsem, wsem):
    wid = _worker_id()
    pltpu.sync_copy(idx.at[pl.ds(wid * _CPW, _CPW)], idx_v)
    iot = jnp.arange(16, dtype=jnp.int32)

    def _transpose_chunk(b):
        rowv = [iot + (b * _CHUNK + 16 * k) for k in range(8)]

        def erow(ee, colv):
            for k in range(8):
                v = plsc.load_gather(gbuf, [rowv[k], colv])
                tbuf[b, ee // 8, ee % 8, pl.ds(16 * k, 16)] = v
            return colv + 1

        lax.fori_loop(0, 32, erow, jnp.zeros((16,), jnp.int32))

    def _run(out, hbase):
        def group(g, carry):
            @pl.when(g > 0)
            def _():
                for b in range(_NBB):
                    pltpu.make_async_copy(
                        tbuf.at[b], out.at[0, :, 0], wsem
                    ).wait()

            for b in range(_NBB):
                jj = g * _NBB + b
                pltpu.async_copy(
                    table.at[idx_v.at[jj]],
                    gbuf.at[pl.ds(b * _CHUNK, _CHUNK)],
                    gsem,
                )
            for b in range(_NBB):
                jj = g * _NBB + b
                pltpu.make_async_copy(
                    table.at[idx_v.at[jj]],
                    gbuf.at[pl.ds(b * _CHUNK, _CHUNK)],
                    gsem,
                ).wait()
            for b in range(_NBB):
                jj = g * _NBB + b
                _transpose_chunk(b)
                h = hbase + jj
                pltpu.async_copy(
                    tbuf.at[b], out.at[h // 32, :, h % 32], wsem
                )
            return carry

        lax.fori_loop(0, _NGB, group, 0)
        for b in range(_NBB):
            pltpu.make_async_copy(tbuf.at[b], out.at[0, :, 0], wsem).wait()

    @pl.when(wid < _NS)
    def _():
        _run(out0, wid * _CPW)

    @pl.when(wid >= _NS)
    def _():
        _run(out1, wid * _CPW - _HALF)


_gather = pl.kernel(
    _gather_body,
    out_type=(
        jax.ShapeDtypeStruct((_MAXLEN, 4, 32, 8, 128), jnp.float32),
        jax.ShapeDtypeStruct((_MAXLEN, 4, 32, 8, 128), jnp.float32),
    ),
    mesh=plsc.VectorSubcoreMesh(core_axis_name="c", subcore_axis_name="s"),
    compiler_params=pltpu.CompilerParams(
        use_tc_tiling_on_sc=False,
        needs_layout_passes=False,
        disable_bounds_checks=True,
    ),
    scratch_types=[
        pltpu.VMEM((_CPW, _CHUNK), jnp.int32),
        pltpu.VMEM((_NBB * _CHUNK, _EMB), jnp.float32),
        pltpu.VMEM((_NBB, 4, 8, 128), jnp.float32),
        pltpu.SemaphoreType.DMA,
        pltpu.SemaphoreType.DMA,
    ],
)


def _seqlen_body(x_ref, o_ref):
    o_ref[...] = jnp.sum(jnp.sign(x_ref[...]), axis=1, keepdims=True)


_seqlen = pl.pallas_call(
    _seqlen_body,
    out_shape=jax.ShapeDtypeStruct((_BATCH * 2, 1), jnp.int32),
)


def kernel(x, embedding_w):
    wpack = lax.optimization_barrier(embedding_w.reshape(_VOCAB // 4, 128))
    wrm = wpack.reshape(_VOCAB, _EMB)
    xt = jnp.transpose(x, (1, 2, 0)).reshape(_CHUNKS, _CHUNK)
    o0, o1 = _gather(wrm, xt)
    e_hypo = o0.transpose(2, 4, 0, 1, 3).reshape(_BATCH, _MAXLEN, _EMB)
    e_prem = o1.transpose(2, 4, 0, 1, 3).reshape(_BATCH, _MAXLEN, _EMB)
    seq = _seqlen(x.reshape(_BATCH * 2, _MAXLEN)).reshape(_BATCH, 2)
    return (e_hypo, e_prem, seq[:, 0], seq[:, 1])


# speedup vs baseline: 1.5392x; 1.0783x over previous
import jax
import jax.numpy as jnp
from jax import lax
from jax.experimental import pallas as pl
from jax.experimental.pallas import tpu as pltpu
from jax.experimental.pallas import tpu_sc as plsc

_VOCAB = 1000000
_EMB = 32
_BATCH = 4096
_MAXLEN = 50

_NC = 2
_NS = 16
_NW = _NC * _NS

_CHUNK = 128
_CHUNKS = 2 * _MAXLEN * (_BATCH // _CHUNK)
_CPW = _CHUNKS // _NW
_NBB = 10
_NGB = _CPW // _NBB
_HALF = _MAXLEN * (_BATCH // _CHUNK)


def _worker_id():
    return lax.axis_index("s") * _NC + lax.axis_index("c")


def _gather_body(table, idx, out0, out1, idx_v, gbuf, tbuf, gsem, wsem):
    wid = _worker_id()
    pltpu.sync_copy(idx.at[pl.ds(wid * _CPW, _CPW)], idx_v)
    iot = jnp.arange(16, dtype=jnp.int32)

    def _transpose_chunk(b):
        rowv = [iot + (b * _CHUNK + 16 * k) for k in range(8)]

        def erow(ee, colv):
            vs = [plsc.load_gather(gbuf, [rowv[k], colv]) for k in range(8)]
            for k in range(8):
                tbuf[b, ee // 8, ee % 8, pl.ds(16 * k, 16)] = vs[k]
            return colv + 1

        lax.fori_loop(0, 32, erow, jnp.zeros((16,), jnp.int32))

    def _run(out, hbase):
        def group(g, carry):
            @pl.when(g > 0)
            def _():
                for b in range(_NBB):
                    pltpu.make_async_copy(
                        tbuf.at[b], out.at[0, :, 0], wsem
                    ).wait()

            for b in range(_NBB):
                jj = g * _NBB + b
                pltpu.async_copy(
                    table.at[idx_v.at[jj]],
                    gbuf.at[pl.ds(b * _CHUNK, _CHUNK)],
                    gsem,
                )
            for b in range(_NBB):
                jj = g * _NBB + b
                pltpu.make_async_copy(
                    table.at[idx_v.at[jj]],
                    gbuf.at[pl.ds(b * _CHUNK, _CHUNK)],
                    gsem,
                ).wait()
            for b in range(_NBB):
                jj = g * _NBB + b
                _transpose_chunk(b)
                h = hbase + jj
                pltpu.async_copy(
                    tbuf.at[b], out.at[h // 32, :, h % 32], wsem
                )
            return carry

        lax.fori_loop(0, _NGB, group, 0)
        for b in range(_NBB):
            pltpu.make_async_copy(tbuf.at[b], out.at[0, :, 0], wsem).wait()

    @pl.when(wid < _NS)
    def _():
        _run(out0, wid * _CPW)

    @pl.when(wid >= _NS)
    def _():
        _run(out1, wid * _CPW - _HALF)


_gather = pl.kernel(
    _gather_body,
    out_type=(
        jax.ShapeDtypeStruct((_MAXLEN, 4, 32, 8, 128), jnp.float32),
        jax.ShapeDtypeStruct((_MAXLEN, 4, 32, 8, 128), jnp.float32),
    ),
    mesh=plsc.VectorSubcoreMesh(core_axis_name="c", subcore_axis_name="s"),
    compiler_params=pltpu.CompilerParams(
        use_tc_tiling_on_sc=False,
        needs_layout_passes=False,
        disable_bounds_checks=True,
    ),
    scratch_types=[
        pltpu.VMEM((_CPW, _CHUNK), jnp.int32),
        pltpu.VMEM((_NBB * _CHUNK, _EMB), jnp.float32),
        pltpu.VMEM((_NBB, 4, 8, 128), jnp.float32),
        pltpu.SemaphoreType.DMA,
        pltpu.SemaphoreType.DMA,
    ],
)


def _seqlen_body(x_ref, o_ref):
    o_ref[...] = jnp.sum(jnp.sign(x_ref[...]), axis=1, keepdims=True)


_seqlen = pl.pallas_call(
    _seqlen_body,
    out_shape=jax.ShapeDtypeStruct((_BATCH * 2, 1), jnp.int32),
)


def kernel(x, embedding_w):
    wpack = lax.optimization_barrier(embedding_w.reshape(_VOCAB // 4, 128))
    wrm = wpack.reshape(_VOCAB, _EMB)
    xt = jnp.transpose(x, (1, 2, 0)).reshape(_CHUNKS, _CHUNK)
    o0, o1 = _gather(wrm, xt)
    e_hypo = o0.transpose(2, 4, 0, 1, 3).reshape(_BATCH, _MAXLEN, _EMB)
    e_prem = o1.transpose(2, 4, 0, 1, 3).reshape(_BATCH, _MAXLEN, _EMB)
    seq = _seqlen(x.reshape(_BATCH * 2, _MAXLEN)).reshape(_BATCH, 2)
    return (e_hypo, e_prem, seq[:, 0], seq[:, 1])


# speedup vs baseline: 1.5494x; 1.0067x over previous
import jax
import jax.numpy as jnp
from jax import lax
from jax.experimental import pallas as pl
from jax.experimental.pallas import tpu as pltpu
from jax.experimental.pallas import tpu_sc as plsc

_VOCAB = 1000000
_EMB = 32
_BATCH = 4096
_MAXLEN = 50

_NC = 2
_NS = 16
_NW = _NC * _NS

_CHUNK = 128
_CHUNKS = 2 * _MAXLEN * (_BATCH // _CHUNK)
_CPW = _CHUNKS // _NW
_NBB = 10
_NGB = _CPW // _NBB
_HALF = _MAXLEN * (_BATCH // _CHUNK)


def _worker_id():
    return lax.axis_index("s") * _NC + lax.axis_index("c")


def _gather_body(table, idx, out0, out1, idx_v, gbuf, tbuf, gsem, wsem):
    wid = _worker_id()
    pltpu.sync_copy(idx.at[pl.ds(wid * _CPW, _CPW)], idx_v)
    iot = jnp.arange(16, dtype=jnp.int32)

    def _transpose_chunk(b):
        rowv = [iot + (b * _CHUNK + 16 * k) for k in range(8)]

        def erow(e2, colv):
            colv1 = colv + 1
            ee = e2 * 2
            vs = [plsc.load_gather(gbuf, [rowv[k], colv]) for k in range(8)]
            ws = [plsc.load_gather(gbuf, [rowv[k], colv1]) for k in range(8)]
            for k in range(8):
                tbuf[b, ee // 8, ee % 8, pl.ds(16 * k, 16)] = vs[k]
            for k in range(8):
                tbuf[b, (ee + 1) // 8, (ee + 1) % 8, pl.ds(16 * k, 16)] = ws[k]
            return colv1 + 1

        lax.fori_loop(0, 16, erow, jnp.zeros((16,), jnp.int32))

    def _run(out, hbase):
        def group(g, carry):
            @pl.when(g > 0)
            def _():
                for b in range(_NBB):
                    pltpu.make_async_copy(
                        tbuf.at[b], out.at[0, :, 0], wsem
                    ).wait()

            for b in range(_NBB):
                jj = g * _NBB + b
                pltpu.async_copy(
                    table.at[idx_v.at[jj]],
                    gbuf.at[pl.ds(b * _CHUNK, _CHUNK)],
                    gsem,
                )
            for b in range(_NBB):
                jj = g * _NBB + b
                pltpu.make_async_copy(
                    table.at[idx_v.at[jj]],
                    gbuf.at[pl.ds(b * _CHUNK, _CHUNK)],
                    gsem,
                ).wait()
            for b in range(_NBB):
                jj = g * _NBB + b
                _transpose_chunk(b)
                h = hbase + jj
                pltpu.async_copy(
                    tbuf.at[b], out.at[h // 32, :, h % 32], wsem
                )
            return carry

        lax.fori_loop(0, _NGB, group, 0)
        for b in range(_NBB):
            pltpu.make_async_copy(tbuf.at[b], out.at[0, :, 0], wsem).wait()

    @pl.when(wid < _NS)
    def _():
        _run(out0, wid * _CPW)

    @pl.when(wid >= _NS)
    def _():
        _run(out1, wid * _CPW - _HALF)


_gather = pl.kernel(
    _gather_body,
    out_type=(
        jax.ShapeDtypeStruct((_MAXLEN, 4, 32, 8, 128), jnp.float32),
        jax.ShapeDtypeStruct((_MAXLEN, 4, 32, 8, 128), jnp.float32),
    ),
    mesh=plsc.VectorSubcoreMesh(core_axis_name="c", subcore_axis_name="s"),
    compiler_params=pltpu.CompilerParams(
        use_tc_tiling_on_sc=False,
        needs_layout_passes=False,
        disable_bounds_checks=True,
    ),
    scratch_types=[
        pltpu.VMEM((_CPW, _CHUNK), jnp.int32),
        pltpu.VMEM((_NBB * _CHUNK, _EMB), jnp.float32),
        pltpu.VMEM((_NBB, 4, 8, 128), jnp.float32),
        pltpu.SemaphoreType.DMA,
        pltpu.SemaphoreType.DMA,
    ],
)


def _seqlen_body(x_ref, o_ref):
    o_ref[...] = jnp.sum(jnp.sign(x_ref[...]), axis=1, keepdims=True)


_seqlen = pl.pallas_call(
    _seqlen_body,
    out_shape=jax.ShapeDtypeStruct((_BATCH * 2, 1), jnp.int32),
)


def kernel(x, embedding_w):
    wpack = lax.optimization_barrier(embedding_w.reshape(_VOCAB // 4, 128))
    wrm = wpack.reshape(_VOCAB, _EMB)
    xt = jnp.transpose(x, (1, 2, 0)).reshape(_CHUNKS, _CHUNK)
    o0, o1 = _gather(wrm, xt)
    e_hypo = o0.transpose(2, 4, 0, 1, 3).reshape(_BATCH, _MAXLEN, _EMB)
    e_prem = o1.transpose(2, 4, 0, 1, 3).reshape(_BATCH, _MAXLEN, _EMB)
    seq = _seqlen(x.reshape(_BATCH * 2, _MAXLEN)).reshape(_BATCH, 2)
    return (e_hypo, e_prem, seq[:, 0], seq[:, 1])


# speedup vs baseline: 1.5861x; 1.0237x over previous
import jax
import jax.numpy as jnp
from jax import lax
from jax.experimental import pallas as pl
from jax.experimental.pallas import tpu as pltpu
from jax.experimental.pallas import tpu_sc as plsc

_VOCAB = 1000000
_EMB = 32
_BATCH = 4096
_MAXLEN = 50

_NC = 2
_NS = 16
_NW = _NC * _NS

_CHUNK = 128
_CHUNKS = 2 * _MAXLEN * (_BATCH // _CHUNK)
_CPW = _CHUNKS // _NW
_NBB = 5
_NGB = _CPW // _NBB
_HALF = _MAXLEN * (_BATCH // _CHUNK)


def _worker_id():
    return lax.axis_index("s") * _NC + lax.axis_index("c")


def _gather_body(
    table, idx, out0, out1, idx_v, gbuf, tbuf, gsem0, gsem1, wsem
):
    wid = _worker_id()
    pltpu.sync_copy(idx.at[pl.ds(wid * _CPW, _CPW)], idx_v)
    iot = jnp.arange(16, dtype=jnp.int32)
    gsems = (gsem0, gsem1)

    def _transpose_chunk(slot, b):
        rowv = [iot + (slot * _CHUNK + 16 * k) for k in range(8)]

        def erow(e2, colv):
            colv1 = colv + 1
            ee = e2 * 2
            vs = [plsc.load_gather(gbuf, [rowv[k], colv]) for k in range(8)]
            ws = [plsc.load_gather(gbuf, [rowv[k], colv1]) for k in range(8)]
            for k in range(8):
                tbuf[b, ee // 8, ee % 8, pl.ds(16 * k, 16)] = vs[k]
            for k in range(8):
                tbuf[b, (ee + 1) // 8, (ee + 1) % 8, pl.ds(16 * k, 16)] = ws[k]
            return colv1 + 1

        lax.fori_loop(0, 16, erow, jnp.zeros((16,), jnp.int32))

    def _run(out, hbase):
        def _fire(g, ph, sem):
            for b in range(_NBB):
                jj = g * _NBB + b
                pltpu.async_copy(
                    table.at[idx_v.at[jj]],
                    gbuf.at[pl.ds((ph * _NBB + b) * _CHUNK, _CHUNK)],
                    sem,
                )

        def _drain(g, ph, sem):
            for b in range(_NBB):
                jj = g * _NBB + b
                pltpu.make_async_copy(
                    table.at[idx_v.at[jj]],
                    gbuf.at[pl.ds((ph * _NBB + b) * _CHUNK, _CHUNK)],
                    sem,
                ).wait()

        _fire(0, 0, gsems[0])

        def group2(g2, carry):
            for ph in range(2):
                g = 2 * g2 + ph

                @pl.when(g + 1 < _NGB)
                def _(ph=ph):
                    _fire(g + 1, 1 - ph, gsems[1 - ph])

                _drain(g, ph, gsems[ph])

                @pl.when(g > 0)
                def _():
                    for b in range(_NBB):
                        pltpu.make_async_copy(
                            tbuf.at[b], out.at[0, :, 0], wsem
                        ).wait()

                for b in range(_NBB):
                    jj = g * _NBB + b
                    _transpose_chunk(ph * _NBB + b, b)
                    h = hbase + jj
                    pltpu.async_copy(
                        tbuf.at[b], out.at[h // 32, :, h % 32], wsem
                    )
            return carry

        lax.fori_loop(0, _NGB // 2, group2, 0)
        for b in range(_NBB):
            pltpu.make_async_copy(tbuf.at[b], out.at[0, :, 0], wsem).wait()

    @pl.when(wid < _NS)
    def _():
        _run(out0, wid * _CPW)

    @pl.when(wid >= _NS)
    def _():
        _run(out1, wid * _CPW - _HALF)


_gather = pl.kernel(
    _gather_body,
    out_type=(
        jax.ShapeDtypeStruct((_MAXLEN, 4, 32, 8, 128), jnp.float32),
        jax.ShapeDtypeStruct((_MAXLEN, 4, 32, 8, 128), jnp.float32),
    ),
    mesh=plsc.VectorSubcoreMesh(core_axis_name="c", subcore_axis_name="s"),
    compiler_params=pltpu.CompilerParams(
        use_tc_tiling_on_sc=False,
        needs_layout_passes=False,
        disable_bounds_checks=True,
    ),
    scratch_types=[
        pltpu.VMEM((_CPW, _CHUNK), jnp.int32),
        pltpu.VMEM((2 * _NBB * _CHUNK, _EMB), jnp.float32),
        pltpu.VMEM((_NBB, 4, 8, 128), jnp.float32),
        pltpu.SemaphoreType.DMA,
        pltpu.SemaphoreType.DMA,
        pltpu.SemaphoreType.DMA,
    ],
)


def _seqlen_body(x_ref, o_ref):
    o_ref[...] = jnp.sum(jnp.sign(x_ref[...]), axis=1, keepdims=True)


_seqlen = pl.pallas_call(
    _seqlen_body,
    out_shape=jax.ShapeDtypeStruct((_BATCH * 2, 1), jnp.int32),
)


def kernel(x, embedding_w):
    wpack = lax.optimization_barrier(embedding_w.reshape(_VOCAB // 4, 128))
    wrm = wpack.reshape(_VOCAB, _EMB)
    xt = jnp.transpose(x, (1, 2, 0)).reshape(_CHUNKS, _CHUNK)
    o0, o1 = _gather(wrm, xt)
    e_hypo = o0.transpose(2, 4, 0, 1, 3).reshape(_BATCH, _MAXLEN, _EMB)
    e_prem = o1.transpose(2, 4, 0, 1, 3).reshape(_BATCH, _MAXLEN, _EMB)
    seq = _seqlen(x.reshape(_BATCH * 2, _MAXLEN)).reshape(_BATCH, 2)
    return (e_hypo, e_prem, seq[:, 0], seq[:, 1])
